# baseline (speedup 1.0000x reference)
import functools

import jax
import jax.numpy as jnp
from jax import lax
from jax.experimental import pallas as pl
from jax.experimental.pallas import tpu as pltpu
from jax.experimental.pallas import tpu_sc as plsc

E = 8
TOP_K = 2
D = 1024
DFF = 4096
T = 2048
NSLOT = T * TOP_K
BLK = 512
NBLK = NSLOT // BLK + E
NPAD = NBLK * BLK
CHUNK = 128

NC, NS, L = 2, 16, 16
NW = NC * NS
TPW = T // NW


def _routing_math(x, gw):
    f32 = jnp.float32
    hi = jax.lax.Precision.HIGHEST
    logits = lax.dot_general(x, gw, (((1,), (1,)), ((), ())),
                             preferred_element_type=f32)

    iota_e = lax.broadcasted_iota(jnp.int32, (T, E), 1)
    v1 = jnp.max(logits, axis=1, keepdims=True)
    a1 = jnp.min(jnp.where(logits == v1, iota_e, 127), axis=1, keepdims=True)
    m1 = iota_e == a1
    l2 = jnp.where(m1, -jnp.inf, logits)
    v2 = jnp.max(l2, axis=1, keepdims=True)
    a2 = jnp.min(jnp.where(l2 == v2, iota_e, 127), axis=1, keepdims=True)
    m2 = iota_e == a2

    g1 = 1.0 / (1.0 + jnp.exp(v2 - v1))
    g2 = 1.0 - g1

    o1 = m1.astype(f32)
    o2 = m2.astype(f32)
    o = o1 + o2

    ii = lax.broadcasted_iota(jnp.int32, (CHUNK, CHUNK), 0)
    jj = lax.broadcasted_iota(jnp.int32, (CHUNK, CHUNK), 1)
    ltri = (jj < ii).astype(f32)
    base = jnp.zeros((1, E), f32)
    cparts = []
    for c in range(T // CHUNK):
        oc = lax.slice(o, (c * CHUNK, 0), ((c + 1) * CHUNK, E))
        pc = lax.dot_general(ltri, oc, (((1,), (0,)), ((), ())),
                             precision=hi, preferred_element_type=f32)
        cparts.append(pc + base)
        base = base + jnp.sum(oc, axis=0, keepdims=True)
    cpre = jnp.concatenate(cparts, axis=0)
    counts = base

    blocks = jnp.floor((counts + (BLK - 1)) * (1.0 / BLK))
    ie = lax.broadcasted_iota(jnp.int32, (E, E), 0)
    je = lax.broadcasted_iota(jnp.int32, (E, E), 1)
    utri = (ie <= je).astype(f32)
    cumb = lax.dot_general(blocks, utri, (((1,), (0,)), ((), ())),
                           precision=hi, preferred_element_type=f32)
    poff = (cumb - blocks) * float(BLK)

    rank1 = jnp.sum(o1 * cpre, axis=1, keepdims=True)
    rank2 = jnp.sum(o2 * cpre, axis=1, keepdims=True)
    off1 = jnp.sum(o1 * poff, axis=1, keepdims=True)
    off2 = jnp.sum(o2 * poff, axis=1, keepdims=True)
    pos1 = (off1 + rank1).astype(jnp.int32)
    pos2 = (off2 + rank2).astype(jnp.int32)

    bb = lax.broadcasted_iota(jnp.int32, (E, NBLK), 1)
    cumb_i = jnp.transpose(cumb).astype(jnp.int32)
    bexp = jnp.sum((bb >= cumb_i).astype(jnp.int32), axis=0,
                   keepdims=True)
    bexp = jnp.minimum(bexp, E - 1)
    nbu = jnp.max(cumb, axis=1, keepdims=True).astype(jnp.int32)

    bprev = jnp.concatenate([bexp[:, :1], bexp[:, :NBLK - 1]], axis=1)
    chg = (bexp != bprev).astype(jnp.int32)
    jn = lax.broadcasted_iota(jnp.int32, (NBLK, NBLK), 1)
    bn = lax.broadcasted_iota(jnp.int32, (NBLK, NBLK), 0)
    live_chg = (jn > bn) & (jnp.broadcast_to(chg, (NBLK, NBLK)) == 1) \
        & (jn < nbu[0, 0])
    cand = jnp.where(live_chg, jn, 2 * NBLK)
    nci = jnp.min(cand, axis=1, keepdims=True)
    pfv = (nci < 2 * NBLK).astype(jnp.int32)
    oh = (jn == jnp.minimum(nci, NBLK - 1)).astype(jnp.int32)
    nexte = jnp.sum(oh * jnp.broadcast_to(bexp, (NBLK, NBLK)), axis=1,
                    keepdims=True)

    return logits, pos1, pos2, g1, g2, bexp, nbu, nexte, pfv


def _routing_body(x_ref, gw_ref, logits_ref, posb_ref, gmat_ref, bexp_ref,
                  nbu_ref, nexte_ref, pfv_ref):
    (logits, pos1, pos2, g1, g2, bexp, nbu, nexte,
     pfv) = _routing_math(x_ref[...], gw_ref[...])
    logits_ref[...] = logits
    posb_ref[0:1, :] = jnp.transpose(pos1)
    posb_ref[1:2, :] = jnp.transpose(pos2)
    gmat_ref[0] = jnp.broadcast_to(g1, (T, L))
    gmat_ref[1] = jnp.broadcast_to(g2, (T, L))
    bexp_ref[...] = bexp
    nbu_ref[...] = nbu
    nexte_ref[...] = nexte
    pfv_ref[...] = pfv


def _routing(x, gw):
    return pl.pallas_call(
        _routing_body,
        out_shape=(
            jax.ShapeDtypeStruct((T, E), jnp.float32),
            jax.ShapeDtypeStruct((2, T), jnp.int32),
            jax.ShapeDtypeStruct((2, T, L), jnp.float32),
            jax.ShapeDtypeStruct((1, NBLK), jnp.int32),
            jax.ShapeDtypeStruct((1, 1), jnp.int32),
            jax.ShapeDtypeStruct((NBLK, 1), jnp.int32),
            jax.ShapeDtypeStruct((NBLK, 1), jnp.int32),
        ),
    )(x, gw)


def _dispatch_body(x_hbm, posb_hbm, xs_hbm, xv, p1v, p2v, sem):
    cid = lax.axis_index("c")
    sid = lax.axis_index("s")
    wid = sid * NC + cid
    base = wid * TPW
    pltpu.sync_copy(posb_hbm.at[0, pl.ds(base, TPW)], p1v)
    pltpu.sync_copy(posb_hbm.at[1, pl.ds(base, TPW)], p2v)
    pltpu.sync_copy(x_hbm.at[pl.ds(base, TPW)], xv)
    pltpu.async_copy(xv, xs_hbm.at[p1v], sem).wait()
    pltpu.async_copy(xv, xs_hbm.at[p2v], sem).wait()


def _dispatch(x, posb):
    mesh = plsc.VectorSubcoreMesh(core_axis_name="c", subcore_axis_name="s")
    return pl.kernel(
        _dispatch_body,
        mesh=mesh,
        out_type=jax.ShapeDtypeStruct((NPAD, D), jnp.float32),
        scratch_types=[
            pltpu.VMEM((TPW, D), jnp.float32),
            pltpu.VMEM((TPW,), jnp.int32),
            pltpu.VMEM((TPW,), jnp.int32),
            pltpu.SemaphoreType.DMA,
        ],
    )(x, posb)


def _erf(z):
    return lax.erf(z)


NCH = 16


def _moe_body(bexp_ref, nbu_ref, nexte_ref, pfv_ref, xs_ref,
              wfc_hbm, wp_hbm, out_ref,
              wfc_v, wp_v, wfc_b, wp_b, sem1, sem2):
    b = pl.program_id(0)
    e = bexp_ref[b]
    prev = bexp_ref[jnp.maximum(b - 1, 0)]
    live = b < nbu_ref[0]
    is_new = jnp.logical_and(live, jnp.logical_or(b == 0, e != prev))

    @pl.when(is_new)
    def _load():
        @pl.when(b == 0)
        def _cold():
            pltpu.make_async_copy(wfc_hbm.at[e], wfc_v, sem1).start()
            pltpu.make_async_copy(wp_hbm.at[e], wp_v, sem2).start()

        pltpu.make_async_copy(wfc_hbm.at[e], wfc_v, sem1).wait()
        pltpu.make_async_copy(wp_hbm.at[e], wp_v, sem2).wait()
        wfc_b[...] = wfc_v[...].astype(jnp.bfloat16)
        wp_b[...] = wp_v[...].astype(jnp.bfloat16)

        @pl.when(pfv_ref[b] == 1)
        def _pf():
            ne = nexte_ref[b]
            pltpu.make_async_copy(wfc_hbm.at[ne], wfc_v, sem1).start()
            pltpu.make_async_copy(wp_hbm.at[ne], wp_v, sem2).start()

    @pl.when(live)
    def _compute():
        xb = xs_ref[...].astype(jnp.bfloat16)
        ck = DFF // NCH
        oT = jnp.zeros((D, BLK), jnp.float32)
        for k in range(NCH):
            hk = lax.dot_general(wfc_b[pl.ds(k * ck, ck), :], xb,
                                 (((1,), (1,)), ((), ())),
                                 preferred_element_type=jnp.float32)
            gk = (0.5 * hk * (1.0 + _erf(hk * 0.7071067811865476))
                  ).astype(jnp.bfloat16)
            oT = oT + lax.dot_general(wp_b[:, pl.ds(k * ck, ck)], gk,
                                      (((1,), (0,)), ((), ())),
                                      preferred_element_type=jnp.float32)
        out_ref[...] = jnp.transpose(oT)


def _grouped_gemm(xs, w_fc, w_proj, bexp, nbu, nexte, pfv):
    grid_spec = pltpu.PrefetchScalarGridSpec(
        num_scalar_prefetch=4,
        grid=(NBLK,),
        in_specs=[
            pl.BlockSpec((BLK, D), lambda b, *_: (b, 0)),
            pl.BlockSpec(memory_space=pltpu.MemorySpace.HBM),
            pl.BlockSpec(memory_space=pltpu.MemorySpace.HBM),
        ],
        out_specs=pl.BlockSpec((BLK, D), lambda b, *_: (b, 0)),
        scratch_shapes=[
            pltpu.VMEM((DFF, D), jnp.float32),
            pltpu.VMEM((D, DFF), jnp.float32),
            pltpu.VMEM((DFF, D), jnp.bfloat16),
            pltpu.VMEM((D, DFF), jnp.bfloat16),
            pltpu.SemaphoreType.DMA,
            pltpu.SemaphoreType.DMA,
        ],
    )
    return pl.pallas_call(
        _moe_body,
        grid_spec=grid_spec,
        out_shape=jax.ShapeDtypeStruct((NPAD, D), jnp.float32),
    )(bexp, nbu, nexte, pfv, xs, w_fc, w_proj)


SUB = 16


NSUB = TPW // SUB


def _combine_body(h_hbm, posb3_hbm, gmat_hbm, out_hbm,
                  b1v, b2v, ov, p1v, p2v, g1v, g2v, sa1, sa2, sb1, sb2):
    cid = lax.axis_index("c")
    sid = lax.axis_index("s")
    wid = sid * NC + cid
    base = wid * TPW
    pltpu.sync_copy(posb3_hbm.at[0, pl.ds(wid * NSUB, NSUB)], p1v)
    pltpu.sync_copy(posb3_hbm.at[1, pl.ds(wid * NSUB, NSUB)], p2v)
    pltpu.sync_copy(gmat_hbm.at[0, pl.ds(base, TPW)], g1v)
    pltpu.sync_copy(gmat_hbm.at[1, pl.ds(base, TPW)], g2v)
    sems = ((sa1, sa2), (sb1, sb2))

    def start(s):
        p = s % 2
        return (pltpu.async_copy(h_hbm.at[p1v.at[s]], b1v.at[p], sems[p][0]),
                pltpu.async_copy(h_hbm.at[p2v.at[s]], b2v.at[p], sems[p][1]))

    cp = start(0)
    for s in range(NSUB):
        nxt = start(s + 1) if s + 1 < NSUB else None
        cp[0].wait()
        cp[1].wait()
        p = s % 2
        for r in range(SUB):
            g1r = g1v[s * SUB + r, :]
            g2r = g2v[s * SUB + r, :]

            def cbody(c, _, p=p, r=r, g1r=g1r, g2r=g2r):
                ov[p, r, pl.ds(c * L, L)] = (
                    g1r * b1v[p, r, pl.ds(c * L, L)]
                    + g2r * b2v[p, r, pl.ds(c * L, L)])
                return 0

            lax.fori_loop(0, D // L, cbody, 0, unroll=4)
        pltpu.sync_copy(ov.at[p], out_hbm.at[pl.ds(base + s * SUB, SUB)])
        cp = nxt


def _combine(h, posb, gmat):
    posb3 = posb.reshape(2, T // SUB, SUB)
    mesh = plsc.VectorSubcoreMesh(core_axis_name="c", subcore_axis_name="s")
    return pl.kernel(
        _combine_body,
        mesh=mesh,
        out_type=jax.ShapeDtypeStruct((T, D), jnp.float32),
        scratch_types=[
            pltpu.VMEM((2, SUB, D), jnp.float32),
            pltpu.VMEM((2, SUB, D), jnp.float32),
            pltpu.VMEM((2, SUB, D), jnp.float32),
            pltpu.VMEM((NSUB, SUB), jnp.int32),
            pltpu.VMEM((NSUB, SUB), jnp.int32),
            pltpu.VMEM((TPW, L), jnp.float32),
            pltpu.VMEM((TPW, L), jnp.float32),
            pltpu.SemaphoreType.DMA,
            pltpu.SemaphoreType.DMA,
            pltpu.SemaphoreType.DMA,
            pltpu.SemaphoreType.DMA,
        ],
    )(h, posb3, gmat)


def kernel(hidden_states, gate_w, w_fc, w_proj):
    orig_shape = hidden_states.shape
    x = hidden_states.reshape(-1, D)
    logits, posb, gmat, bexp2d, nbu2d, nexte2d, pfv2d = _routing(x, gate_w)
    bexp = bexp2d.reshape(NBLK)
    nbu = nbu2d.reshape(1)
    nexte = nexte2d.reshape(NBLK)
    pfv = pfv2d.reshape(NBLK)
    xs = _dispatch(x, posb)
    h = _grouped_gemm(xs, w_fc, w_proj, bexp, nbu, nexte, pfv)
    out = _combine(h, posb, gmat)
    return (out.reshape(orig_shape), logits)


# speedup vs baseline: 1.2402x; 1.2402x over previous
import functools

import jax
import jax.numpy as jnp
from jax import lax
from jax.experimental import pallas as pl
from jax.experimental.pallas import tpu as pltpu
from jax.experimental.pallas import tpu_sc as plsc

E = 8
TOP_K = 2
D = 1024
DFF = 4096
T = 2048
NSLOT = T * TOP_K
BLK = 256
NBLK = NSLOT // BLK + E
NPAD = NBLK * BLK
CHUNK = 256

NC, NS, L = 2, 16, 16
NW = NC * NS
TPW = T // NW


def _routing_math(x, gw):
    f32 = jnp.float32
    hi = jax.lax.Precision.HIGHEST
    logits = lax.dot_general(x, gw, (((1,), (1,)), ((), ())),
                             preferred_element_type=f32)

    iota_e = lax.broadcasted_iota(jnp.int32, (T, E), 1)
    v1 = jnp.max(logits, axis=1, keepdims=True)
    a1 = jnp.min(jnp.where(logits == v1, iota_e, 127), axis=1, keepdims=True)
    m1 = iota_e == a1
    l2 = jnp.where(m1, -jnp.inf, logits)
    v2 = jnp.max(l2, axis=1, keepdims=True)
    a2 = jnp.min(jnp.where(l2 == v2, iota_e, 127), axis=1, keepdims=True)
    m2 = iota_e == a2

    g1 = 1.0 / (1.0 + jnp.exp(v2 - v1))
    g2 = 1.0 - g1

    o1 = m1.astype(f32)
    o2 = m2.astype(f32)
    o = o1 + o2

    ii = lax.broadcasted_iota(jnp.int32, (CHUNK, CHUNK), 0)
    jj = lax.broadcasted_iota(jnp.int32, (CHUNK, CHUNK), 1)
    ltri = (jj < ii).astype(f32)
    base = jnp.zeros((1, E), f32)
    cparts = []
    for c in range(T // CHUNK):
        oc = lax.slice(o, (c * CHUNK, 0), ((c + 1) * CHUNK, E))
        pc = lax.dot_general(ltri, oc, (((1,), (0,)), ((), ())),
                             precision=hi, preferred_element_type=f32)
        cparts.append(pc + base)
        base = base + jnp.sum(oc, axis=0, keepdims=True)
    cpre = jnp.concatenate(cparts, axis=0)
    counts = base

    blocks = jnp.floor((counts + (BLK - 1)) * (1.0 / BLK))
    ie = lax.broadcasted_iota(jnp.int32, (E, E), 0)
    je = lax.broadcasted_iota(jnp.int32, (E, E), 1)
    utri = (ie <= je).astype(f32)
    cumb = lax.dot_general(blocks, utri, (((1,), (0,)), ((), ())),
                           precision=hi, preferred_element_type=f32)
    poff = (cumb - blocks) * float(BLK)

    rank1 = jnp.sum(o1 * cpre, axis=1, keepdims=True)
    rank2 = jnp.sum(o2 * cpre, axis=1, keepdims=True)
    off1 = jnp.sum(o1 * poff, axis=1, keepdims=True)
    off2 = jnp.sum(o2 * poff, axis=1, keepdims=True)
    pos1 = (off1 + rank1).astype(jnp.int32)
    pos2 = (off2 + rank2).astype(jnp.int32)

    bb = lax.broadcasted_iota(jnp.int32, (E, NBLK), 1)
    cumb_i = jnp.transpose(cumb).astype(jnp.int32)
    bexp = jnp.sum((bb >= cumb_i).astype(jnp.int32), axis=0,
                   keepdims=True)
    bexp = jnp.minimum(bexp, E - 1)
    nbu = jnp.max(cumb, axis=1, keepdims=True).astype(jnp.int32)

    bprev = jnp.concatenate([bexp[:, :1], bexp[:, :NBLK - 1]], axis=1)
    chg = (bexp != bprev).astype(jnp.int32)
    jn = lax.broadcasted_iota(jnp.int32, (NBLK, NBLK), 1)
    bn = lax.broadcasted_iota(jnp.int32, (NBLK, NBLK), 0)
    live_chg = (jn > bn) & (jnp.broadcast_to(chg, (NBLK, NBLK)) == 1) \
        & (jn < nbu[0, 0])
    cand = jnp.where(live_chg, jn, 2 * NBLK)
    nci = jnp.min(cand, axis=1, keepdims=True)
    pfv = (nci < 2 * NBLK).astype(jnp.int32)
    oh = (jn == jnp.minimum(nci, NBLK - 1)).astype(jnp.int32)
    nexte = jnp.sum(oh * jnp.broadcast_to(bexp, (NBLK, NBLK)), axis=1,
                    keepdims=True)

    return logits, pos1, pos2, g1, g2, bexp, nbu, nexte, pfv


def _routing_body(x_ref, gw_ref, logits_ref, posb_ref, gmat_ref, bexp_ref,
                  nbu_ref, nexte_ref, pfv_ref):
    (logits, pos1, pos2, g1, g2, bexp, nbu, nexte,
     pfv) = _routing_math(x_ref[...], gw_ref[...])
    logits_ref[...] = logits
    posb_ref[0:1, :] = jnp.transpose(pos1)
    posb_ref[1:2, :] = jnp.transpose(pos2)
    gmat_ref[0] = jnp.broadcast_to(g1, (T, L))
    gmat_ref[1] = jnp.broadcast_to(g2, (T, L))
    bexp_ref[...] = bexp
    nbu_ref[...] = nbu
    nexte_ref[...] = nexte
    pfv_ref[...] = pfv


def _routing(x, gw):
    return pl.pallas_call(
        _routing_body,
        out_shape=(
            jax.ShapeDtypeStruct((T, E), jnp.float32),
            jax.ShapeDtypeStruct((2, T), jnp.int32),
            jax.ShapeDtypeStruct((2, T, L), jnp.float32),
            jax.ShapeDtypeStruct((1, NBLK), jnp.int32),
            jax.ShapeDtypeStruct((1, 1), jnp.int32),
            jax.ShapeDtypeStruct((NBLK, 1), jnp.int32),
            jax.ShapeDtypeStruct((NBLK, 1), jnp.int32),
        ),
    )(x, gw)


def _dispatch_body(x_hbm, posb_hbm, xs_hbm, xv, p1v, p2v, sem):
    cid = lax.axis_index("c")
    sid = lax.axis_index("s")
    wid = sid * NC + cid
    base = wid * TPW
    pltpu.sync_copy(posb_hbm.at[0, pl.ds(base, TPW)], p1v)
    pltpu.sync_copy(posb_hbm.at[1, pl.ds(base, TPW)], p2v)
    pltpu.sync_copy(x_hbm.at[pl.ds(base, TPW)], xv)
    pltpu.async_copy(xv, xs_hbm.at[p1v], sem).wait()
    pltpu.async_copy(xv, xs_hbm.at[p2v], sem).wait()


def _dispatch(x, posb):
    mesh = plsc.VectorSubcoreMesh(core_axis_name="c", subcore_axis_name="s")
    return pl.kernel(
        _dispatch_body,
        mesh=mesh,
        out_type=jax.ShapeDtypeStruct((NPAD, D), jnp.float32),
        scratch_types=[
            pltpu.VMEM((TPW, D), jnp.float32),
            pltpu.VMEM((TPW,), jnp.int32),
            pltpu.VMEM((TPW,), jnp.int32),
            pltpu.SemaphoreType.DMA,
        ],
    )(x, posb)


def _erf(z):
    return lax.erf(z)


NCH = 16


def _moe_body(bexp_ref, nbu_ref, nexte_ref, pfv_ref, xs_ref,
              wfc_hbm, wp_hbm, out_ref,
              wfc_v, wp_v, wfc_b, wp_b, sem1, sem2):
    b = pl.program_id(0)
    e = bexp_ref[b]
    prev = bexp_ref[jnp.maximum(b - 1, 0)]
    live = b < nbu_ref[0]
    is_new = jnp.logical_and(live, jnp.logical_or(b == 0, e != prev))

    @pl.when(is_new)
    def _load():
        @pl.when(b == 0)
        def _cold():
            pltpu.make_async_copy(wfc_hbm.at[e], wfc_v, sem1).start()
            pltpu.make_async_copy(wp_hbm.at[e], wp_v, sem2).start()

        pltpu.make_async_copy(wfc_hbm.at[e], wfc_v, sem1).wait()
        pltpu.make_async_copy(wp_hbm.at[e], wp_v, sem2).wait()
        wfc_b[...] = wfc_v[...].astype(jnp.bfloat16)
        wp_b[...] = wp_v[...].astype(jnp.bfloat16)

        @pl.when(pfv_ref[b] == 1)
        def _pf():
            ne = nexte_ref[b]
            pltpu.make_async_copy(wfc_hbm.at[ne], wfc_v, sem1).start()
            pltpu.make_async_copy(wp_hbm.at[ne], wp_v, sem2).start()

    @pl.when(live)
    def _compute():
        xb = xs_ref[...].astype(jnp.bfloat16)
        ck = DFF // NCH
        oT = jnp.zeros((D, BLK), jnp.float32)
        for k in range(NCH):
            hk = lax.dot_general(wfc_b[pl.ds(k * ck, ck), :], xb,
                                 (((1,), (1,)), ((), ())),
                                 preferred_element_type=jnp.float32)
            gk = (0.5 * hk * (1.0 + _erf(hk * 0.7071067811865476))
                  ).astype(jnp.bfloat16)
            oT = oT + lax.dot_general(wp_b[:, pl.ds(k * ck, ck)], gk,
                                      (((1,), (0,)), ((), ())),
                                      preferred_element_type=jnp.float32)
        out_ref[...] = jnp.transpose(oT)


def _grouped_gemm(xs, w_fc, w_proj, bexp, nbu, nexte, pfv):
    grid_spec = pltpu.PrefetchScalarGridSpec(
        num_scalar_prefetch=4,
        grid=(NBLK,),
        in_specs=[
            pl.BlockSpec((BLK, D), lambda b, *_: (b, 0)),
            pl.BlockSpec(memory_space=pltpu.MemorySpace.HBM),
            pl.BlockSpec(memory_space=pltpu.MemorySpace.HBM),
        ],
        out_specs=pl.BlockSpec((BLK, D), lambda b, *_: (b, 0)),
        scratch_shapes=[
            pltpu.VMEM((DFF, D), jnp.float32),
            pltpu.VMEM((D, DFF), jnp.float32),
            pltpu.VMEM((DFF, D), jnp.bfloat16),
            pltpu.VMEM((D, DFF), jnp.bfloat16),
            pltpu.SemaphoreType.DMA,
            pltpu.SemaphoreType.DMA,
        ],
    )
    return pl.pallas_call(
        _moe_body,
        grid_spec=grid_spec,
        out_shape=jax.ShapeDtypeStruct((NPAD, D), jnp.float32),
    )(bexp, nbu, nexte, pfv, xs, w_fc, w_proj)


SUB = 16


NSUB = TPW // SUB


def _combine_body(h_hbm, posb3_hbm, gmat_hbm, out_hbm,
                  b1v, b2v, ov, p1v, p2v, g1v, g2v, sa1, sa2, sb1, sb2,
                  so1, so2):
    cid = lax.axis_index("c")
    sid = lax.axis_index("s")
    wid = sid * NC + cid
    base = wid * TPW
    pltpu.sync_copy(posb3_hbm.at[0, pl.ds(wid * NSUB, NSUB)], p1v)
    pltpu.sync_copy(posb3_hbm.at[1, pl.ds(wid * NSUB, NSUB)], p2v)
    pltpu.sync_copy(gmat_hbm.at[0, pl.ds(base, TPW)], g1v)
    pltpu.sync_copy(gmat_hbm.at[1, pl.ds(base, TPW)], g2v)
    sems = ((sa1, sa2), (sb1, sb2))

    def start(s):
        p = s % 2
        return (pltpu.async_copy(h_hbm.at[p1v.at[s]], b1v.at[p], sems[p][0]),
                pltpu.async_copy(h_hbm.at[p2v.at[s]], b2v.at[p], sems[p][1]))

    osems = (so1, so2)
    cp = start(0)
    outcps = [None, None]
    for s in range(NSUB):
        nxt = start(s + 1) if s + 1 < NSUB else None
        cp[0].wait()
        cp[1].wait()
        p = s % 2
        if outcps[p] is not None:
            outcps[p].wait()
        for r in range(SUB):
            g1r = g1v[s * SUB + r, :]
            g2r = g2v[s * SUB + r, :]

            def cbody(c, _, p=p, r=r, g1r=g1r, g2r=g2r):
                ov[p, r, pl.ds(c * L, L)] = (
                    g1r * b1v[p, r, pl.ds(c * L, L)]
                    + g2r * b2v[p, r, pl.ds(c * L, L)])
                return 0

            lax.fori_loop(0, D // L, cbody, 0, unroll=4)
        outcps[p] = pltpu.async_copy(
            ov.at[p], out_hbm.at[pl.ds(base + s * SUB, SUB)], osems[p])
        cp = nxt
    for oc in outcps:
        if oc is not None:
            oc.wait()


def _combine(h, posb, gmat):
    posb3 = posb.reshape(2, T // SUB, SUB)
    mesh = plsc.VectorSubcoreMesh(core_axis_name="c", subcore_axis_name="s")
    return pl.kernel(
        _combine_body,
        mesh=mesh,
        out_type=jax.ShapeDtypeStruct((T, D), jnp.float32),
        scratch_types=[
            pltpu.VMEM((2, SUB, D), jnp.float32),
            pltpu.VMEM((2, SUB, D), jnp.float32),
            pltpu.VMEM((2, SUB, D), jnp.float32),
            pltpu.VMEM((NSUB, SUB), jnp.int32),
            pltpu.VMEM((NSUB, SUB), jnp.int32),
            pltpu.VMEM((TPW, L), jnp.float32),
            pltpu.VMEM((TPW, L), jnp.float32),
            pltpu.SemaphoreType.DMA,
            pltpu.SemaphoreType.DMA,
            pltpu.SemaphoreType.DMA,
            pltpu.SemaphoreType.DMA,
            pltpu.SemaphoreType.DMA,
            pltpu.SemaphoreType.DMA,
        ],
    )(h, posb3, gmat)


def kernel(hidden_states, gate_w, w_fc, w_proj):
    orig_shape = hidden_states.shape
    x = hidden_states.reshape(-1, D)
    logits, posb, gmat, bexp2d, nbu2d, nexte2d, pfv2d = _routing(x, gate_w)
    bexp = bexp2d.reshape(NBLK)
    nbu = nbu2d.reshape(1)
    nexte = nexte2d.reshape(NBLK)
    pfv = pfv2d.reshape(NBLK)
    xs = _dispatch(x, posb)
    h = _grouped_gemm(xs, w_fc, w_proj, bexp, nbu, nexte, pfv)
    out = _combine(h, posb, gmat)
    return (out.reshape(orig_shape), logits)


# speedup vs baseline: 1.2587x; 1.0149x over previous
import functools

import jax
import jax.numpy as jnp
from jax import lax
from jax.experimental import pallas as pl
from jax.experimental.pallas import tpu as pltpu
from jax.experimental.pallas import tpu_sc as plsc

E = 8
TOP_K = 2
D = 1024
DFF = 4096
T = 2048
NSLOT = T * TOP_K
BLK = 256
NBLK = NSLOT // BLK + E
NPAD = NBLK * BLK
CHUNK = 128

NC, NS, L = 2, 16, 16
NW = NC * NS
TPW = T // NW


def _routing_math(x, gw):
    f32 = jnp.float32
    hi = jax.lax.Precision.HIGHEST
    logits = lax.dot_general(x, gw, (((1,), (1,)), ((), ())),
                             preferred_element_type=f32)

    iota_e = lax.broadcasted_iota(jnp.int32, (T, E), 1)
    v1 = jnp.max(logits, axis=1, keepdims=True)
    a1 = jnp.min(jnp.where(logits == v1, iota_e, 127), axis=1, keepdims=True)
    m1 = iota_e == a1
    l2 = jnp.where(m1, -jnp.inf, logits)
    v2 = jnp.max(l2, axis=1, keepdims=True)
    a2 = jnp.min(jnp.where(l2 == v2, iota_e, 127), axis=1, keepdims=True)
    m2 = iota_e == a2

    g1 = 1.0 / (1.0 + jnp.exp(v2 - v1))
    g2 = 1.0 - g1

    o1 = m1.astype(f32)
    o2 = m2.astype(f32)
    o = o1 + o2

    ii = lax.broadcasted_iota(jnp.int32, (CHUNK, CHUNK), 0)
    jj = lax.broadcasted_iota(jnp.int32, (CHUNK, CHUNK), 1)
    ltri = (jj < ii).astype(f32)
    base = jnp.zeros((1, E), f32)
    cparts = []
    for c in range(T // CHUNK):
        oc = lax.slice(o, (c * CHUNK, 0), ((c + 1) * CHUNK, E))
        pc = lax.dot_general(ltri, oc, (((1,), (0,)), ((), ())),
                             precision=hi, preferred_element_type=f32)
        cparts.append(pc + base)
        base = base + jnp.sum(oc, axis=0, keepdims=True)
    cpre = jnp.concatenate(cparts, axis=0)
    counts = base

    blocks = jnp.floor((counts + (BLK - 1)) * (1.0 / BLK))
    ie = lax.broadcasted_iota(jnp.int32, (E, E), 0)
    je = lax.broadcasted_iota(jnp.int32, (E, E), 1)
    utri = (ie <= je).astype(f32)
    cumb = lax.dot_general(blocks, utri, (((1,), (0,)), ((), ())),
                           precision=hi, preferred_element_type=f32)
    poff = (cumb - blocks) * float(BLK)

    rank1 = jnp.sum(o1 * cpre, axis=1, keepdims=True)
    rank2 = jnp.sum(o2 * cpre, axis=1, keepdims=True)
    off1 = jnp.sum(o1 * poff, axis=1, keepdims=True)
    off2 = jnp.sum(o2 * poff, axis=1, keepdims=True)
    pos1 = (off1 + rank1).astype(jnp.int32)
    pos2 = (off2 + rank2).astype(jnp.int32)

    bb = lax.broadcasted_iota(jnp.int32, (E, NBLK), 1)
    cumb_i = jnp.transpose(cumb).astype(jnp.int32)
    bexp = jnp.sum((bb >= cumb_i).astype(jnp.int32), axis=0,
                   keepdims=True)
    bexp = jnp.minimum(bexp, E - 1)
    nbu = jnp.max(cumb, axis=1, keepdims=True).astype(jnp.int32)

    bprev = jnp.concatenate([bexp[:, :1], bexp[:, :NBLK - 1]], axis=1)
    chg = (bexp != bprev).astype(jnp.int32)
    jn = lax.broadcasted_iota(jnp.int32, (NBLK, NBLK), 1)
    bn = lax.broadcasted_iota(jnp.int32, (NBLK, NBLK), 0)
    live_chg = (jn > bn) & (jnp.broadcast_to(chg, (NBLK, NBLK)) == 1) \
        & (jn < nbu[0, 0])
    cand = jnp.where(live_chg, jn, 2 * NBLK)
    nci = jnp.min(cand, axis=1, keepdims=True)
    pfv = (nci < 2 * NBLK).astype(jnp.int32)
    oh = (jn == jnp.minimum(nci, NBLK - 1)).astype(jnp.int32)
    nexte = jnp.sum(oh * jnp.broadcast_to(bexp, (NBLK, NBLK)), axis=1,
                    keepdims=True)

    return logits, pos1, pos2, g1, g2, bexp, nbu, nexte, pfv


def _routing_body(x_ref, gw_ref, logits_ref, posb_ref, gmat_ref, bexp_ref,
                  nbu_ref, nexte_ref, pfv_ref):
    (logits, pos1, pos2, g1, g2, bexp, nbu, nexte,
     pfv) = _routing_math(x_ref[...], gw_ref[...])
    logits_ref[...] = logits
    posb_ref[0:1, :] = jnp.transpose(pos1)
    posb_ref[1:2, :] = jnp.transpose(pos2)
    gmat_ref[0] = jnp.broadcast_to(g1, (T, L))
    gmat_ref[1] = jnp.broadcast_to(g2, (T, L))
    bexp_ref[...] = bexp
    nbu_ref[...] = nbu
    nexte_ref[...] = nexte
    pfv_ref[...] = pfv


def _routing(x, gw):
    return pl.pallas_call(
        _routing_body,
        out_shape=(
            jax.ShapeDtypeStruct((T, E), jnp.float32),
            jax.ShapeDtypeStruct((2, T), jnp.int32),
            jax.ShapeDtypeStruct((2, T, L), jnp.float32),
            jax.ShapeDtypeStruct((1, NBLK), jnp.int32),
            jax.ShapeDtypeStruct((1, 1), jnp.int32),
            jax.ShapeDtypeStruct((NBLK, 1), jnp.int32),
            jax.ShapeDtypeStruct((NBLK, 1), jnp.int32),
        ),
    )(x, gw)


def _dispatch_body(x_hbm, posb_hbm, xs_hbm, xv, p1v, p2v, sem):
    cid = lax.axis_index("c")
    sid = lax.axis_index("s")
    wid = sid * NC + cid
    base = wid * TPW
    pltpu.sync_copy(posb_hbm.at[0, pl.ds(base, TPW)], p1v)
    pltpu.sync_copy(posb_hbm.at[1, pl.ds(base, TPW)], p2v)
    pltpu.sync_copy(x_hbm.at[pl.ds(base, TPW)], xv)
    pltpu.async_copy(xv, xs_hbm.at[p1v], sem).wait()
    pltpu.async_copy(xv, xs_hbm.at[p2v], sem).wait()


def _dispatch(x, posb):
    mesh = plsc.VectorSubcoreMesh(core_axis_name="c", subcore_axis_name="s")
    return pl.kernel(
        _dispatch_body,
        mesh=mesh,
        out_type=jax.ShapeDtypeStruct((NPAD, D), jnp.float32),
        scratch_types=[
            pltpu.VMEM((TPW, D), jnp.float32),
            pltpu.VMEM((TPW,), jnp.int32),
            pltpu.VMEM((TPW,), jnp.int32),
            pltpu.SemaphoreType.DMA,
        ],
    )(x, posb)


def _erf(z):
    return lax.erf(z)


NCH = 16


def _moe_body(bexp_ref, nbu_ref, nexte_ref, pfv_ref, xs_ref,
              wfc_hbm, wp_hbm, out_ref,
              wfc_v, wp_v, wfc_b, wp_b, sem1, sem2):
    b = pl.program_id(0)
    e = bexp_ref[b]
    prev = bexp_ref[jnp.maximum(b - 1, 0)]
    live = b < nbu_ref[0]
    is_new = jnp.logical_and(live, jnp.logical_or(b == 0, e != prev))

    @pl.when(is_new)
    def _load():
        @pl.when(b == 0)
        def _cold():
            pltpu.make_async_copy(wfc_hbm.at[e], wfc_v, sem1).start()
            pltpu.make_async_copy(wp_hbm.at[e], wp_v, sem2).start()

        pltpu.make_async_copy(wfc_hbm.at[e], wfc_v, sem1).wait()
        pltpu.make_async_copy(wp_hbm.at[e], wp_v, sem2).wait()
        wfc_b[...] = wfc_v[...].astype(jnp.bfloat16)
        wp_b[...] = wp_v[...].astype(jnp.bfloat16)

        @pl.when(pfv_ref[b] == 1)
        def _pf():
            ne = nexte_ref[b]
            pltpu.make_async_copy(wfc_hbm.at[ne], wfc_v, sem1).start()
            pltpu.make_async_copy(wp_hbm.at[ne], wp_v, sem2).start()

    @pl.when(live)
    def _compute():
        xb = xs_ref[...].astype(jnp.bfloat16)
        ck = DFF // NCH
        oT = jnp.zeros((D, BLK), jnp.float32)
        for k in range(NCH):
            hk = lax.dot_general(wfc_b[pl.ds(k * ck, ck), :], xb,
                                 (((1,), (1,)), ((), ())),
                                 preferred_element_type=jnp.float32)
            gk = (0.5 * hk * (1.0 + _erf(hk * 0.7071067811865476))
                  ).astype(jnp.bfloat16)
            oT = oT + lax.dot_general(wp_b[:, pl.ds(k * ck, ck)], gk,
                                      (((1,), (0,)), ((), ())),
                                      preferred_element_type=jnp.float32)
        out_ref[...] = jnp.transpose(oT)


def _grouped_gemm(xs, w_fc, w_proj, bexp, nbu, nexte, pfv):
    grid_spec = pltpu.PrefetchScalarGridSpec(
        num_scalar_prefetch=4,
        grid=(NBLK,),
        in_specs=[
            pl.BlockSpec((BLK, D), lambda b, *_: (b, 0)),
            pl.BlockSpec(memory_space=pltpu.MemorySpace.HBM),
            pl.BlockSpec(memory_space=pltpu.MemorySpace.HBM),
        ],
        out_specs=pl.BlockSpec((BLK, D), lambda b, *_: (b, 0)),
        scratch_shapes=[
            pltpu.VMEM((DFF, D), jnp.float32),
            pltpu.VMEM((D, DFF), jnp.float32),
            pltpu.VMEM((DFF, D), jnp.bfloat16),
            pltpu.VMEM((D, DFF), jnp.bfloat16),
            pltpu.SemaphoreType.DMA,
            pltpu.SemaphoreType.DMA,
        ],
    )
    return pl.pallas_call(
        _moe_body,
        grid_spec=grid_spec,
        out_shape=jax.ShapeDtypeStruct((NPAD, D), jnp.float32),
    )(bexp, nbu, nexte, pfv, xs, w_fc, w_proj)


SUB = 16


NSUB = TPW // SUB


def _combine_body(h_hbm, posb3_hbm, gmat_hbm, out_hbm,
                  b1v, b2v, ov, p1v, p2v, g1v, g2v, sa1, sa2, sb1, sb2,
                  so1, so2):
    cid = lax.axis_index("c")
    sid = lax.axis_index("s")
    wid = sid * NC + cid
    base = wid * TPW
    pltpu.sync_copy(posb3_hbm.at[0, pl.ds(wid * NSUB, NSUB)], p1v)
    pltpu.sync_copy(posb3_hbm.at[1, pl.ds(wid * NSUB, NSUB)], p2v)
    pltpu.sync_copy(gmat_hbm.at[0, pl.ds(base, TPW)], g1v)
    pltpu.sync_copy(gmat_hbm.at[1, pl.ds(base, TPW)], g2v)
    sems = ((sa1, sa2), (sb1, sb2))

    def start(s):
        p = s % 2
        return (pltpu.async_copy(h_hbm.at[p1v.at[s]], b1v.at[p], sems[p][0]),
                pltpu.async_copy(h_hbm.at[p2v.at[s]], b2v.at[p], sems[p][1]))

    osems = (so1, so2)
    cp = start(0)
    outcps = [None, None]
    for s in range(NSUB):
        nxt = start(s + 1) if s + 1 < NSUB else None
        cp[0].wait()
        cp[1].wait()
        p = s % 2
        if outcps[p] is not None:
            outcps[p].wait()
        for r in range(SUB):
            g1r = g1v[s * SUB + r, :]
            g2r = g2v[s * SUB + r, :]

            def cbody(c, _, p=p, r=r, g1r=g1r, g2r=g2r):
                ov[p, r, pl.ds(c * L, L)] = (
                    g1r * b1v[p, r, pl.ds(c * L, L)]
                    + g2r * b2v[p, r, pl.ds(c * L, L)])
                return 0

            lax.fori_loop(0, D // L, cbody, 0, unroll=4)
        outcps[p] = pltpu.async_copy(
            ov.at[p], out_hbm.at[pl.ds(base + s * SUB, SUB)], osems[p])
        cp = nxt
    for oc in outcps:
        if oc is not None:
            oc.wait()


def _combine(h, posb, gmat):
    posb3 = posb.reshape(2, T // SUB, SUB)
    mesh = plsc.VectorSubcoreMesh(core_axis_name="c", subcore_axis_name="s")
    return pl.kernel(
        _combine_body,
        mesh=mesh,
        out_type=jax.ShapeDtypeStruct((T, D), jnp.float32),
        scratch_types=[
            pltpu.VMEM((2, SUB, D), jnp.float32),
            pltpu.VMEM((2, SUB, D), jnp.float32),
            pltpu.VMEM((2, SUB, D), jnp.float32),
            pltpu.VMEM((NSUB, SUB), jnp.int32),
            pltpu.VMEM((NSUB, SUB), jnp.int32),
            pltpu.VMEM((TPW, L), jnp.float32),
            pltpu.VMEM((TPW, L), jnp.float32),
            pltpu.SemaphoreType.DMA,
            pltpu.SemaphoreType.DMA,
            pltpu.SemaphoreType.DMA,
            pltpu.SemaphoreType.DMA,
            pltpu.SemaphoreType.DMA,
            pltpu.SemaphoreType.DMA,
        ],
    )(h, posb3, gmat)


def kernel(hidden_states, gate_w, w_fc, w_proj):
    orig_shape = hidden_states.shape
    x = hidden_states.reshape(-1, D)
    logits, posb, gmat, bexp2d, nbu2d, nexte2d, pfv2d = _routing(x, gate_w)
    bexp = bexp2d.reshape(NBLK)
    nbu = nbu2d.reshape(1)
    nexte = nexte2d.reshape(NBLK)
    pfv = pfv2d.reshape(NBLK)
    xs = _dispatch(x, posb)
    h = _grouped_gemm(xs, w_fc, w_proj, bexp, nbu, nexte, pfv)
    out = _combine(h, posb, gmat)
    return (out.reshape(orig_shape), logits)
